# Initial kernel scaffold; baseline (speedup 1.0000x reference)
import jax
import jax.numpy as jnp
from jax.experimental import pallas as pl


def kernel(input, tables):
    raise NotImplementedError("write your pallas kernel here")


# speedup vs baseline: 3.8340x; 3.8340x over previous
import functools

import jax
import jax.numpy as jnp
from jax import lax
from jax.experimental import pallas as pl
from jax.experimental.pallas import tpu as pltpu
from jax.experimental.pallas import tpu_sc as plsc

NUM_HEADS = 4
ROWS_PER_TABLE = 100000
DIM = 64
LANES = 16
NC, NS = 2, 16
NW = NC * NS
CHUNK = 256
IDX_ROW = 128
ROWS_PER_CHUNK = CHUNK * NUM_HEADS // IDX_ROW


def _mhe_sc(idx2d, tab_flat, B):
    ntok_w = B // NW
    nchunks = ntok_w // CHUNK
    mesh = plsc.VectorSubcoreMesh(core_axis_name="c", subcore_axis_name="s")

    @functools.partial(
        pl.kernel,
        out_type=jax.ShapeDtypeStruct((B, DIM), jnp.float32),
        mesh=mesh,
        compiler_params=pltpu.CompilerParams(use_tc_tiling_on_sc=False),
        scratch_types=[
            pltpu.VMEM((ROWS_PER_CHUNK, IDX_ROW), jnp.int32),
            pltpu.VMEM((CHUNK * NUM_HEADS, DIM), jnp.float32),
            pltpu.VMEM((CHUNK, DIM), jnp.float32),
            pltpu.SemaphoreType.DMA,
        ],
    )
    def k(idx_hbm, tab_hbm, out_hbm, idx_v, buf_v, out_v, sem):
        wid = lax.axis_index("s") * NC + lax.axis_index("c")
        offs = (lax.iota(jnp.int32, LANES) & (NUM_HEADS - 1)) * ROWS_PER_TABLE

        def chunk_body(ci, carry):
            tok0 = pl.multiple_of(wid * ntok_w + ci * CHUNK, CHUNK)
            row0 = pl.multiple_of(tok0 * NUM_HEADS // IDX_ROW, ROWS_PER_CHUNK)
            pltpu.sync_copy(idx_hbm.at[pl.ds(row0, ROWS_PER_CHUNK)], idx_v)
            for j in range(ROWS_PER_CHUNK):
                for kk in range(IDX_ROW // LANES):
                    sl = pl.ds(kk * LANES, LANES)
                    idx_v[j, sl] = idx_v[j, sl] + offs
            cps = [
                pltpu.async_copy(
                    tab_hbm.at[idx_v.at[j]],
                    buf_v.at[pl.ds(j * IDX_ROW, IDX_ROW)],
                    sem,
                )
                for j in range(ROWS_PER_CHUNK)
            ]
            for cp in cps:
                cp.wait()

            def acc_body(t, c2):
                r = t * NUM_HEADS
                for d in range(DIM // LANES):
                    sl = pl.ds(d * LANES, LANES)
                    out_v[t, sl] = (buf_v[r, sl] + buf_v[r + 1, sl]) + (
                        buf_v[r + 2, sl] + buf_v[r + 3, sl]
                    )
                return c2

            lax.fori_loop(0, CHUNK, acc_body, 0)
            pltpu.sync_copy(out_v, out_hbm.at[pl.ds(tok0, CHUNK)])
            return carry

        lax.fori_loop(0, nchunks, chunk_body, 0)

    return k(idx2d, tab_flat)


def kernel(input, tables):
    bd, t, h = input.shape
    B = bd * t
    idx2d = input.astype(jnp.int32).reshape(B * h // IDX_ROW, IDX_ROW)
    tab_flat = tables.reshape(h * ROWS_PER_TABLE, DIM)
    out = _mhe_sc(idx2d, tab_flat, B)
    return out.reshape(bd, t, DIM)


# speedup vs baseline: 4.3164x; 1.1258x over previous
import functools

import jax
import jax.numpy as jnp
from jax import lax
from jax.experimental import pallas as pl
from jax.experimental.pallas import tpu as pltpu
from jax.experimental.pallas import tpu_sc as plsc

NUM_HEADS = 4
ROWS_PER_TABLE = 100000
DIM = 64
LANES = 16
NC, NS = 2, 16
NW = NC * NS
CHUNK = 128
IDX_ROW = 128
ROWS_PER_CHUNK = CHUNK * NUM_HEADS // IDX_ROW


def _mhe_sc(idx2d, tab_flat, B):
    ntok_w = B // NW
    nchunks = ntok_w // CHUNK
    mesh = plsc.VectorSubcoreMesh(core_axis_name="c", subcore_axis_name="s")

    @functools.partial(
        pl.kernel,
        out_type=jax.ShapeDtypeStruct((B, DIM), jnp.float32),
        mesh=mesh,
        compiler_params=pltpu.CompilerParams(use_tc_tiling_on_sc=False),
        scratch_types=[
            pltpu.VMEM((ROWS_PER_CHUNK, IDX_ROW), jnp.int32),
            pltpu.VMEM((ROWS_PER_CHUNK, IDX_ROW), jnp.int32),
            pltpu.VMEM((CHUNK * NUM_HEADS, DIM), jnp.float32),
            pltpu.VMEM((CHUNK * NUM_HEADS, DIM), jnp.float32),
            pltpu.VMEM((CHUNK, DIM), jnp.float32),
            pltpu.VMEM((CHUNK, DIM), jnp.float32),
            pltpu.SemaphoreType.DMA,
            pltpu.SemaphoreType.DMA,
            pltpu.SemaphoreType.DMA,
            pltpu.SemaphoreType.DMA,
        ],
    )
    def k(idx_hbm, tab_hbm, out_hbm, idx_a, idx_b, buf_a, buf_b, out_a,
          out_b, gsem_a, gsem_b, osem_a, osem_b):
        wid = lax.axis_index("s") * NC + lax.axis_index("c")
        offs = (lax.iota(jnp.int32, LANES) & (NUM_HEADS - 1)) * ROWS_PER_TABLE

        def fire(idx_v, buf_v, gsem, tok0):
            row0 = pl.multiple_of(
                tok0 * NUM_HEADS // IDX_ROW, ROWS_PER_CHUNK)
            pltpu.sync_copy(idx_hbm.at[pl.ds(row0, ROWS_PER_CHUNK)], idx_v)
            for j in range(ROWS_PER_CHUNK):
                for kk in range(IDX_ROW // LANES):
                    sl = pl.ds(kk * LANES, LANES)
                    idx_v[j, sl] = idx_v[j, sl] + offs
            for j in range(ROWS_PER_CHUNK):
                pltpu.async_copy(
                    tab_hbm.at[idx_v.at[j]],
                    buf_v.at[pl.ds(j * IDX_ROW, IDX_ROW)],
                    gsem,
                )

        def drain(idx_v, buf_v, gsem):
            for j in range(ROWS_PER_CHUNK):
                pltpu.make_async_copy(
                    tab_hbm.at[idx_v.at[j]],
                    buf_v.at[pl.ds(j * IDX_ROW, IDX_ROW)],
                    gsem,
                ).wait()

        def consume(buf_v, out_v, osem, tok0, wait_prev):
            @pl.when(wait_prev)
            def _():
                pltpu.make_async_copy(
                    out_v, out_hbm.at[pl.ds(tok0, CHUNK)], osem).wait()

            def acc_body(t, c2):
                r = t * NUM_HEADS
                for d in range(DIM // LANES):
                    sl = pl.ds(d * LANES, LANES)
                    out_v[t, sl] = (buf_v[r, sl] + buf_v[r + 1, sl]) + (
                        buf_v[r + 2, sl] + buf_v[r + 3, sl]
                    )
                return c2

            lax.fori_loop(0, CHUNK, acc_body, 0)
            pltpu.async_copy(out_v, out_hbm.at[pl.ds(tok0, CHUNK)], osem)

        tok_base = wid * ntok_w
        fire(idx_a, buf_a, gsem_a, pl.multiple_of(tok_base, CHUNK))

        def body(g, carry):
            c0 = pl.multiple_of(tok_base + 2 * g * CHUNK, CHUNK)
            c1 = pl.multiple_of(c0 + CHUNK, CHUNK)
            fire(idx_b, buf_b, gsem_b, c1)
            drain(idx_a, buf_a, gsem_a)
            consume(buf_a, out_a, osem_a, c0, g > 0)

            @pl.when(2 * g + 2 < nchunks)
            def _():
                fire(idx_a, buf_a, gsem_a, pl.multiple_of(c0 + 2 * CHUNK,
                                                          CHUNK))

            drain(idx_b, buf_b, gsem_b)
            consume(buf_b, out_b, osem_b, c1, g > 0)
            return carry

        lax.fori_loop(0, nchunks // 2, body, 0)
        pltpu.make_async_copy(
            out_a, out_hbm.at[pl.ds(tok_base, CHUNK)], osem_a).wait()
        pltpu.make_async_copy(
            out_b, out_hbm.at[pl.ds(tok_base, CHUNK)], osem_b).wait()

    return k(idx2d, tab_flat)


def kernel(input, tables):
    bd, t, h = input.shape
    B = bd * t
    idx2d = input.astype(jnp.int32).reshape(B * h // IDX_ROW, IDX_ROW)
    tab_flat = tables.reshape(h * ROWS_PER_TABLE, DIM)
    out = _mhe_sc(idx2d, tab_flat, B)
    return out.reshape(bd, t, DIM)
